# baseline (speedup 1.0000x reference)
import jax
import jax.numpy as jnp
from jax.experimental import pallas as pl

NUM_CLASSES = 1000
CPAD = 1024
B, F = 4096, 26
ROWS = B * F
ROW_TILE = 2048


def _onehot_block(idx_ref, out_ref):
    idx = idx_ref[...]
    classes = jax.lax.broadcasted_iota(jnp.int32, (ROW_TILE, CPAD), 1)
    out_ref[...] = (idx == classes).astype(jnp.int8)


def kernel(input):
    idx = input.astype(jnp.int32).reshape(ROWS, 1)
    out8 = pl.pallas_call(
        _onehot_block,
        grid=(ROWS // ROW_TILE,),
        in_specs=[pl.BlockSpec((ROW_TILE, 1), lambda i: (i, 0))],
        out_specs=pl.BlockSpec((ROW_TILE, CPAD), lambda i: (i, 0)),
        out_shape=jax.ShapeDtypeStruct((ROWS, CPAD), jnp.int8),
    )(idx)
    out = out8[:, :NUM_CLASSES].reshape(B, F, NUM_CLASSES).astype(jnp.float32)
    return out


# speedup vs baseline: 1.5499x; 1.5499x over previous
import jax
import jax.numpy as jnp
from jax.experimental import pallas as pl

NUM_CLASSES = 1000
B, F = 4096, 26
B_TILE = 128


def _onehot_block(idx_ref, out_ref):
    idx = idx_ref[...]
    classes = jax.lax.broadcasted_iota(
        jnp.int32, (B_TILE, F, NUM_CLASSES), 2
    )
    out_ref[...] = (idx == classes).astype(jnp.int8)


def kernel(input):
    idx = input.astype(jnp.int32)
    out8 = pl.pallas_call(
        _onehot_block,
        grid=(B // B_TILE,),
        in_specs=[pl.BlockSpec((B_TILE, F, 1), lambda i: (i, 0, 0))],
        out_specs=pl.BlockSpec((B_TILE, F, NUM_CLASSES), lambda i: (i, 0, 0)),
        out_shape=jax.ShapeDtypeStruct((B, F, NUM_CLASSES), jnp.int8),
    )(idx)
    return out8.astype(jnp.float32)


# speedup vs baseline: 1.5755x; 1.0166x over previous
import jax
import jax.numpy as jnp
from jax.experimental import pallas as pl

NUM_CLASSES = 1000
B, F = 4096, 26
B_TILE = 512


def _onehot_block(idx_ref, out_ref):
    idx = idx_ref[...]
    classes = jax.lax.broadcasted_iota(
        jnp.int32, (B_TILE, F, NUM_CLASSES), 2
    )
    out_ref[...] = (idx == classes).astype(jnp.int8)


def kernel(input):
    idx = input.astype(jnp.int32)
    out8 = pl.pallas_call(
        _onehot_block,
        grid=(B // B_TILE,),
        in_specs=[pl.BlockSpec((B_TILE, F, 1), lambda i: (i, 0, 0))],
        out_specs=pl.BlockSpec((B_TILE, F, NUM_CLASSES), lambda i: (i, 0, 0)),
        out_shape=jax.ShapeDtypeStruct((B, F, NUM_CLASSES), jnp.int8),
    )(idx)
    return out8.astype(jnp.float32)
